# Initial kernel scaffold; baseline (speedup 1.0000x reference)
import jax
import jax.numpy as jnp
from jax.experimental import pallas as pl


def kernel(x, edge_index, W1, b1, W2, b2):
    raise NotImplementedError("write your pallas kernel here")


# speedup vs baseline: 10.4792x; 10.4792x over previous
import functools

import jax
import jax.numpy as jnp
from jax import lax
from jax.experimental import pallas as pl
from jax.experimental.pallas import tpu as pltpu
from jax.experimental.pallas import tpu_sc as plsc

N = 10000
E = 320000
D = 128
NC = 2
NS = 16
NW = NC * NS
LANES = 16
EB = 80
TPR = E // EB // NW
NCK = 5
CKB = TPR // NCK
NPAD = 10240
RPT = NPAD // NS
RCH = 128
WCH = RPT // RCH

_MESH = dict(core_axis_name="c", subcore_axis_name="s", num_cores=NC,
             num_subcores=NS)


def _zero_vmem(ref, nrows):
    def zr(i, carry):
        for t16 in range(D // LANES):
            ref[i, pl.ds(t16 * LANES, LANES)] = jnp.zeros((LANES,),
                                                          jnp.float32)
        return carry

    lax.fori_loop(0, nrows, zr, 0)


def _sc_degrees(er):
    @functools.partial(
        pl.kernel,
        out_type=jax.ShapeDtypeStruct((NC, NS * WCH, RCH, D), jnp.float32),
        mesh=plsc.VectorSubcoreMesh(**_MESH),
        scratch_types=[
            pltpu.VMEM((CKB, EB), jnp.int32),
            pltpu.VMEM((EB, D), jnp.float32),
            pltpu.VMEM((RCH, D), jnp.float32),
            pltpu.VMEM_SHARED((NPAD, D), jnp.float32),
        ],
    )
    def k(er_hbm, deg_out, idx_v, ones_v, stage_v, acc_sh):
        c = lax.axis_index("c")
        s = lax.axis_index("s")

        def fill_ones(i, carry):
            for t16 in range(D // LANES):
                ones_v[i, pl.ds(t16 * LANES, LANES)] = jnp.ones(
                    (LANES,), jnp.float32)
            return carry

        lax.fori_loop(0, EB, fill_ones, 0)
        _zero_vmem(stage_v, RCH)
        for t in range(WCH):
            off = pl.multiple_of(s * RPT + t * RCH, RCH)
            pltpu.sync_copy(stage_v, acc_sh.at[pl.ds(off, RCH)])
        plsc.subcore_barrier()

        for p in range(2):
            for q in range(NCK):
                pltpu.sync_copy(er_hbm.at[c, 2 * s + p, q], idx_v)

                def add_batch(j, carry):
                    pltpu.sync_copy(ones_v, acc_sh.at[idx_v.at[j]], add=True)
                    return carry

                lax.fori_loop(0, CKB, add_batch, 0)
        plsc.subcore_barrier()
        for t in range(WCH):
            off = pl.multiple_of(s * RPT + t * RCH, RCH)
            pltpu.sync_copy(acc_sh.at[pl.ds(off, RCH)], stage_v)
            pltpu.sync_copy(stage_v, deg_out.at[c, s * WCH + t])

    return k(er)


def _sc_edge_agg(er, h):
    @functools.partial(
        pl.kernel,
        out_type=jax.ShapeDtypeStruct((NC, NS * WCH, RCH, D), jnp.float32),
        mesh=plsc.VectorSubcoreMesh(**_MESH),
        scratch_types=[
            pltpu.VMEM((CKB, EB), jnp.int32),
            pltpu.VMEM((CKB, EB), jnp.int32),
            pltpu.VMEM((EB, D), jnp.float32),
            pltpu.VMEM((RCH, D), jnp.float32),
            pltpu.VMEM_SHARED((NPAD, D), jnp.float32),
            pltpu.SemaphoreType.DMA,
        ],
    )
    def k(er_hbm, h_hbm, out_hbm, src_v, dst_v, rows_v, stage_v, acc_sh, gsem):
        c = lax.axis_index("c")
        s = lax.axis_index("s")
        w = c * NS + s

        _zero_vmem(stage_v, RCH)
        for t in range(WCH):
            off = pl.multiple_of(s * RPT + t * RCH, RCH)
            pltpu.sync_copy(stage_v, acc_sh.at[pl.ds(off, RCH)])
        plsc.subcore_barrier()

        for q in range(NCK):
            pltpu.sync_copy(er_hbm.at[0, w, q], src_v)
            pltpu.sync_copy(er_hbm.at[1, w, q], dst_v)

            def edge_batch(j, carry):
                pltpu.async_copy(h_hbm.at[src_v.at[j]], rows_v, gsem).wait()
                pltpu.sync_copy(rows_v, acc_sh.at[dst_v.at[j]], add=True)
                return carry

            lax.fori_loop(0, CKB, edge_batch, 0)
        plsc.subcore_barrier()
        for t in range(WCH):
            off = pl.multiple_of(s * RPT + t * RCH, RCH)
            pltpu.sync_copy(acc_sh.at[pl.ds(off, RCH)], stage_v)
            pltpu.sync_copy(stage_v, out_hbm.at[c, s * WCH + t])

    return k(er, h)


def _norm_col(deg_block):
    return jnp.where(deg_block > 0.0,
                     lax.rsqrt(jnp.maximum(deg_block, 1.0)), 0.0)[:, 0:1]


_RB = 1000


def _tc_layer1(x, W1, degs):
    def body(x_ref, w_ref, ds_ref, o_ref):
        ns = _norm_col(ds_ref[...])
        h = jnp.dot(x_ref[...], w_ref[...], preferred_element_type=jnp.float32)
        o_ref[...] = h * ns

    return pl.pallas_call(
        body,
        grid=(N // _RB,),
        in_specs=[
            pl.BlockSpec((_RB, D), lambda i: (i, 0)),
            pl.BlockSpec((D, D), lambda i: (0, 0)),
            pl.BlockSpec((_RB, D), lambda i: (i, 0)),
        ],
        out_specs=pl.BlockSpec((_RB, D), lambda i: (i, 0)),
        out_shape=jax.ShapeDtypeStruct((N, D), jnp.float32),
    )(x, W1, degs)


def _tc_layer2(a0, a1, degs, degd, b1r, W2):
    def body(a0_ref, a1_ref, ds_ref, dd_ref, b_ref, w_ref, o_ref):
        nd = _norm_col(dd_ref[...])
        ns = _norm_col(ds_ref[...])
        y = jnp.maximum((a0_ref[...] + a1_ref[...]) * nd + b_ref[...], 0.0)
        o_ref[...] = jnp.dot(
            y, w_ref[...], preferred_element_type=jnp.float32) * ns

    return pl.pallas_call(
        body,
        grid=(N // _RB,),
        in_specs=[
            pl.BlockSpec((_RB, D), lambda i: (i, 0)),
            pl.BlockSpec((_RB, D), lambda i: (i, 0)),
            pl.BlockSpec((_RB, D), lambda i: (i, 0)),
            pl.BlockSpec((_RB, D), lambda i: (i, 0)),
            pl.BlockSpec((1, D), lambda i: (0, 0)),
            pl.BlockSpec((D, D), lambda i: (0, 0)),
        ],
        out_specs=pl.BlockSpec((_RB, D), lambda i: (i, 0)),
        out_shape=jax.ShapeDtypeStruct((N, D), jnp.float32),
    )(a0, a1, degs, degd, b1r, W2)


def _tc_final(a0, a1, degd, b2r):
    def body(a0_ref, a1_ref, dd_ref, b_ref, o_ref):
        nd = _norm_col(dd_ref[...])
        o_ref[...] = (a0_ref[...] + a1_ref[...]) * nd + b_ref[...]

    return pl.pallas_call(
        body,
        grid=(N // _RB,),
        in_specs=[
            pl.BlockSpec((_RB, D), lambda i: (i, 0)),
            pl.BlockSpec((_RB, D), lambda i: (i, 0)),
            pl.BlockSpec((_RB, D), lambda i: (i, 0)),
            pl.BlockSpec((1, D), lambda i: (0, 0)),
        ],
        out_specs=pl.BlockSpec((_RB, D), lambda i: (i, 0)),
        out_shape=jax.ShapeDtypeStruct((N, D), jnp.float32),
    )(a0, a1, degd, b2r)


def kernel(x, edge_index, W1, b1, W2, b2):
    er = edge_index.reshape(2, NW, NCK, CKB, EB)
    deg = _sc_degrees(er).reshape(NC, NPAD, D)[:, :N]
    degs, degd = deg[0], deg[1]
    hs1 = _tc_layer1(x, W1, degs)
    ap1 = _sc_edge_agg(er, hs1).reshape(NC, NPAD, D)[:, :N]
    hs2 = _tc_layer2(ap1[0], ap1[1], degs, degd, b1.reshape(1, D), W2)
    ap2 = _sc_edge_agg(er, hs2).reshape(NC, NPAD, D)[:, :N]
    return _tc_final(ap2[0], ap2[1], degd, b2.reshape(1, D))


# speedup vs baseline: 12.2987x; 1.1736x over previous
import functools

import jax
import jax.numpy as jnp
from jax import lax
from jax.experimental import pallas as pl
from jax.experimental.pallas import tpu as pltpu
from jax.experimental.pallas import tpu_sc as plsc

N = 10000
E = 320000
D = 128
NC = 2
NS = 16
NW = NC * NS
LANES = 16
EB = 80
TPR = E // EB // NW
NCK = 5
CKB = TPR // NCK
NPAD = 10240
RPT = NPAD // NS

_MESH = dict(core_axis_name="c", subcore_axis_name="s", num_cores=NC,
             num_subcores=NS)


def _sc_degrees(er, zrows):
    @functools.partial(
        pl.kernel,
        out_type=jax.ShapeDtypeStruct((NC, NS, RPT, D), jnp.float32),
        mesh=plsc.VectorSubcoreMesh(**_MESH),
        scratch_types=[
            pltpu.VMEM((CKB, EB), jnp.int32),
            pltpu.VMEM((EB, D), jnp.float32),
            pltpu.VMEM_SHARED((NPAD, D), jnp.float32),
        ],
    )
    def k(er_hbm, z_hbm, deg_out, idx_v, ones_v, acc_sh):
        c = lax.axis_index("c")
        s = lax.axis_index("s")

        def fill_ones(i, carry):
            for t16 in range(D // LANES):
                ones_v[i, pl.ds(t16 * LANES, LANES)] = jnp.ones(
                    (LANES,), jnp.float32)
            return carry

        lax.fori_loop(0, EB, fill_ones, 0)
        off = pl.multiple_of(s * RPT, RPT)
        pltpu.sync_copy(z_hbm, acc_sh.at[pl.ds(off, RPT)])
        plsc.subcore_barrier()

        for p in range(2):
            for q in range(NCK):
                pltpu.sync_copy(er_hbm.at[c, 2 * s + p, q], idx_v)

                def add_batch(j, carry):
                    pltpu.sync_copy(ones_v, acc_sh.at[idx_v.at[j]], add=True)
                    return carry

                lax.fori_loop(0, CKB, add_batch, 0)
        plsc.subcore_barrier()
        pltpu.sync_copy(acc_sh.at[pl.ds(off, RPT)], deg_out.at[c, s])

    return k(er, zrows)


def _sc_edge_agg(er, h, zrows):
    @functools.partial(
        pl.kernel,
        out_type=jax.ShapeDtypeStruct((NC, NS, RPT, D), jnp.float32),
        mesh=plsc.VectorSubcoreMesh(**_MESH),
        scratch_types=[
            pltpu.VMEM((CKB, EB), jnp.int32),
            pltpu.VMEM((CKB, EB), jnp.int32),
            pltpu.VMEM((CKB, EB), jnp.int32),
            pltpu.VMEM((CKB, EB), jnp.int32),
            pltpu.VMEM((EB, D), jnp.float32),
            pltpu.VMEM((EB, D), jnp.float32),
            pltpu.SemaphoreType.DMA,
            pltpu.SemaphoreType.DMA,
            pltpu.SemaphoreType.DMA,
            pltpu.SemaphoreType.DMA,
            pltpu.SemaphoreType.DMA,
            pltpu.VMEM_SHARED((NPAD, D), jnp.float32),
        ],
    )
    def k(er_hbm, h_hbm, z_hbm, out_hbm, src0, src1, dst0, dst1,
          rows0, rows1, gs0, gs1, ss0, ss1, isem, acc_sh):
        c = lax.axis_index("c")
        s = lax.axis_index("s")
        w = c * NS + s
        srcb, dstb = (src0, src1), (dst0, dst1)
        rows, gsems, ssems = (rows0, rows1), (gs0, gs1), (ss0, ss1)

        off = pl.multiple_of(s * RPT, RPT)
        pltpu.sync_copy(z_hbm, acc_sh.at[pl.ds(off, RPT)])
        plsc.subcore_barrier()

        def start_gather(sref, j, b):
            pltpu.async_copy(h_hbm.at[sref.at[j]], rows[b], gsems[b])

        def wait_gather(sref, j, b):
            pltpu.make_async_copy(h_hbm.at[sref.at[j]], rows[b],
                                  gsems[b]).wait()

        def start_scatter(dref, j, b):
            pltpu.async_copy(rows[b], acc_sh.at[dref.at[j]], ssems[b],
                             add=True)

        def wait_scatter(dref, j, b):
            pltpu.make_async_copy(rows[b], acc_sh.at[dref.at[j]],
                                  ssems[b]).wait()

        for q in range(NCK):
            qb = q % 2
            sref, dref = srcb[qb], dstb[qb]
            if q == 0:
                pltpu.sync_copy(er_hbm.at[0, w, 0], sref)
                pltpu.sync_copy(er_hbm.at[1, w, 0], dref)
            else:
                pltpu.make_async_copy(er_hbm.at[0, w, q], sref, isem).wait()
                pltpu.make_async_copy(er_hbm.at[1, w, q], dref, isem).wait()
            if q + 1 < NCK:
                nb = (q + 1) % 2
                pltpu.async_copy(er_hbm.at[0, w, q + 1], srcb[nb], isem)
                pltpu.async_copy(er_hbm.at[1, w, q + 1], dstb[nb], isem)

            start_gather(sref, 0, 0)
            wait_gather(sref, 0, 0)
            start_scatter(dref, 0, 0)
            start_gather(sref, 1, 1)

            def pair(jj, carry):
                j1 = 2 * jj + 1
                wait_gather(sref, j1, 1)
                start_scatter(dref, j1, 1)
                wait_scatter(dref, j1 - 1, 0)
                start_gather(sref, j1 + 1, 0)
                j2 = 2 * jj + 2
                wait_gather(sref, j2, 0)
                start_scatter(dref, j2, 0)
                wait_scatter(dref, j2 - 1, 1)
                start_gather(sref, j2 + 1, 1)
                return carry

            lax.fori_loop(0, (CKB - 3) // 2, pair, 0)
            wait_gather(sref, CKB - 2, 1)
            start_scatter(dref, CKB - 2, 1)
            wait_scatter(dref, CKB - 3, 0)
            start_gather(sref, CKB - 1, 0)
            wait_gather(sref, CKB - 1, 0)
            start_scatter(dref, CKB - 1, 0)
            wait_scatter(dref, CKB - 2, 1)
            wait_scatter(dref, CKB - 1, 0)

        plsc.subcore_barrier()
        pltpu.sync_copy(acc_sh.at[pl.ds(off, RPT)], out_hbm.at[c, s])

    return k(er, h, zrows)


def _norm_col(deg_block):
    return jnp.where(deg_block > 0.0,
                     lax.rsqrt(jnp.maximum(deg_block, 1.0)), 0.0)[:, 0:1]


_RB = 1000


def _tc_layer1(x, W1, degs):
    def body(x_ref, w_ref, ds_ref, o_ref):
        ns = _norm_col(ds_ref[...])
        h = jnp.dot(x_ref[...], w_ref[...], preferred_element_type=jnp.float32)
        o_ref[...] = h * ns

    return pl.pallas_call(
        body,
        grid=(N // _RB,),
        in_specs=[
            pl.BlockSpec((_RB, D), lambda i: (i, 0)),
            pl.BlockSpec((D, D), lambda i: (0, 0)),
            pl.BlockSpec((_RB, D), lambda i: (i, 0)),
        ],
        out_specs=pl.BlockSpec((_RB, D), lambda i: (i, 0)),
        out_shape=jax.ShapeDtypeStruct((N, D), jnp.float32),
    )(x, W1, degs)


def _tc_layer2(a0, a1, degs, degd, b1r, W2):
    def body(a0_ref, a1_ref, ds_ref, dd_ref, b_ref, w_ref, o_ref):
        nd = _norm_col(dd_ref[...])
        ns = _norm_col(ds_ref[...])
        y = jnp.maximum((a0_ref[...] + a1_ref[...]) * nd + b_ref[...], 0.0)
        o_ref[...] = jnp.dot(
            y, w_ref[...], preferred_element_type=jnp.float32) * ns

    return pl.pallas_call(
        body,
        grid=(N // _RB,),
        in_specs=[
            pl.BlockSpec((_RB, D), lambda i: (i, 0)),
            pl.BlockSpec((_RB, D), lambda i: (i, 0)),
            pl.BlockSpec((_RB, D), lambda i: (i, 0)),
            pl.BlockSpec((_RB, D), lambda i: (i, 0)),
            pl.BlockSpec((1, D), lambda i: (0, 0)),
            pl.BlockSpec((D, D), lambda i: (0, 0)),
        ],
        out_specs=pl.BlockSpec((_RB, D), lambda i: (i, 0)),
        out_shape=jax.ShapeDtypeStruct((N, D), jnp.float32),
    )(a0, a1, degs, degd, b1r, W2)


def _tc_final(a0, a1, degd, b2r):
    def body(a0_ref, a1_ref, dd_ref, b_ref, o_ref):
        nd = _norm_col(dd_ref[...])
        o_ref[...] = (a0_ref[...] + a1_ref[...]) * nd + b_ref[...]

    return pl.pallas_call(
        body,
        grid=(N // _RB,),
        in_specs=[
            pl.BlockSpec((_RB, D), lambda i: (i, 0)),
            pl.BlockSpec((_RB, D), lambda i: (i, 0)),
            pl.BlockSpec((_RB, D), lambda i: (i, 0)),
            pl.BlockSpec((1, D), lambda i: (0, 0)),
        ],
        out_specs=pl.BlockSpec((_RB, D), lambda i: (i, 0)),
        out_shape=jax.ShapeDtypeStruct((N, D), jnp.float32),
    )(a0, a1, degd, b2r)


def kernel(x, edge_index, W1, b1, W2, b2):
    er = edge_index.reshape(2, NW, NCK, CKB, EB)
    zrows = jnp.zeros((RPT, D), jnp.float32)
    deg = _sc_degrees(er, zrows).reshape(NC, NPAD, D)[:, :N]
    degs, degd = deg[0], deg[1]
    hs1 = _tc_layer1(x, W1, degs)
    ap1 = _sc_edge_agg(er, hs1, zrows).reshape(NC, NPAD, D)[:, :N]
    hs2 = _tc_layer2(ap1[0], ap1[1], degs, degd, b1.reshape(1, D), W2)
    ap2 = _sc_edge_agg(er, hs2, zrows).reshape(NC, NPAD, D)[:, :N]
    return _tc_final(ap2[0], ap2[1], degd, b2.reshape(1, D))


# speedup vs baseline: 15.2647x; 1.2412x over previous
import functools

import jax
import jax.numpy as jnp
from jax import lax
from jax.experimental import pallas as pl
from jax.experimental.pallas import tpu as pltpu
from jax.experimental.pallas import tpu_sc as plsc

N = 10000
E = 320000
D = 128
NC = 2
NS = 16
NW = NC * NS
LANES = 16
EB = 80
TPR = E // EB // NW
NCK = 5
CKB = TPR // NCK
NPAD = 10240
RPT = NPAD // NS

_MESH = dict(core_axis_name="c", subcore_axis_name="s", num_cores=NC,
             num_subcores=NS)


def _sc_degrees(er, zrows):
    @functools.partial(
        pl.kernel,
        out_type=jax.ShapeDtypeStruct((NC, NS, RPT, D), jnp.float32),
        mesh=plsc.VectorSubcoreMesh(**_MESH),
        scratch_types=[
            pltpu.VMEM((CKB, EB), jnp.int32),
            pltpu.VMEM((EB, D), jnp.float32),
            pltpu.VMEM_SHARED((NPAD, D), jnp.float32),
        ],
    )
    def k(er_hbm, z_hbm, deg_out, idx_v, ones_v, acc_sh):
        c = lax.axis_index("c")
        s = lax.axis_index("s")

        def fill_ones(i, carry):
            for t16 in range(D // LANES):
                ones_v[i, pl.ds(t16 * LANES, LANES)] = jnp.ones(
                    (LANES,), jnp.float32)
            return carry

        lax.fori_loop(0, EB, fill_ones, 0)
        off = pl.multiple_of(s * RPT, RPT)
        pltpu.sync_copy(z_hbm, acc_sh.at[pl.ds(off, RPT)])
        plsc.subcore_barrier()

        for p in range(2):
            for q in range(NCK):
                pltpu.sync_copy(er_hbm.at[c, 2 * s + p, q], idx_v)

                def add_batch(j, carry):
                    pltpu.sync_copy(ones_v, acc_sh.at[idx_v.at[j]], add=True)
                    return carry

                lax.fori_loop(0, CKB, add_batch, 0)
        plsc.subcore_barrier()
        pltpu.sync_copy(acc_sh.at[pl.ds(off, RPT)], deg_out.at[c, s])

    return k(er, zrows)


def _sc_edge_agg(er, h, zrows):
    @functools.partial(
        pl.kernel,
        out_type=jax.ShapeDtypeStruct((NC, NS, RPT, D), jnp.float32),
        mesh=plsc.VectorSubcoreMesh(**_MESH),
        scratch_types=[
            pltpu.VMEM((CKB, EB), jnp.int32),
            pltpu.VMEM((CKB, EB), jnp.int32),
            pltpu.VMEM((CKB, EB), jnp.int32),
            pltpu.VMEM((CKB, EB), jnp.int32),
            pltpu.VMEM((EB, D), jnp.float32),
            pltpu.VMEM((EB, D), jnp.float32),
            pltpu.VMEM((EB, D), jnp.float32),
            pltpu.SemaphoreType.DMA,
            pltpu.SemaphoreType.DMA,
            pltpu.SemaphoreType.DMA,
            pltpu.SemaphoreType.DMA,
            pltpu.SemaphoreType.DMA,
            pltpu.SemaphoreType.DMA,
            pltpu.SemaphoreType.DMA,
            pltpu.VMEM_SHARED((NPAD, D), jnp.float32),
        ],
    )
    def k(er_hbm, h_hbm, z_hbm, out_hbm, src0, src1, dst0, dst1,
          rows0, rows1, rows2, gs0, gs1, gs2, ss0, ss1, ss2, isem, acc_sh):
        c = lax.axis_index("c")
        s = lax.axis_index("s")
        w = c * NS + s
        srcb, dstb = (src0, src1), (dst0, dst1)
        rows = (rows0, rows1, rows2)
        gsems, ssems = (gs0, gs1, gs2), (ss0, ss1, ss2)

        off = pl.multiple_of(s * RPT, RPT)
        pltpu.sync_copy(z_hbm, acc_sh.at[pl.ds(off, RPT)])
        plsc.subcore_barrier()

        def start_gather(sref, j, b):
            pltpu.async_copy(h_hbm.at[sref.at[j]], rows[b], gsems[b])

        def wait_gather(sref, j, b):
            pltpu.make_async_copy(h_hbm.at[sref.at[j]], rows[b],
                                  gsems[b]).wait()

        def start_scatter(dref, j, b):
            pltpu.async_copy(rows[b], acc_sh.at[dref.at[j]], ssems[b],
                             add=True)

        def wait_scatter(dref, j, b):
            pltpu.make_async_copy(rows[b], acc_sh.at[dref.at[j]],
                                  ssems[b]).wait()

        for q in range(NCK):
            qb = q % 2
            sref, dref = srcb[qb], dstb[qb]
            if q == 0:
                pltpu.sync_copy(er_hbm.at[0, w, 0], sref)
                pltpu.sync_copy(er_hbm.at[1, w, 0], dref)
            else:
                pltpu.make_async_copy(er_hbm.at[0, w, q], sref, isem).wait()
                pltpu.make_async_copy(er_hbm.at[1, w, q], dref, isem).wait()
            if q + 1 < NCK:
                nb = (q + 1) % 2
                pltpu.async_copy(er_hbm.at[0, w, q + 1], srcb[nb], isem)
                pltpu.async_copy(er_hbm.at[1, w, q + 1], dstb[nb], isem)

            start_gather(sref, 0, 0)
            start_gather(sref, 1, 1)
            wait_gather(sref, 0, 0)
            start_scatter(dref, 0, 0)
            start_gather(sref, 2, 2)

            def triple(jj, carry):
                for b in (1, 2, 0):
                    j = 3 * jj + (1 if b == 1 else (2 if b == 2 else 3))
                    wait_gather(sref, j, b)
                    start_scatter(dref, j, b)
                    wait_scatter(dref, j - 1, (b + 2) % 3)
                    start_gather(sref, j + 2, (b + 2) % 3)
                return carry

            lax.fori_loop(0, (CKB - 4) // 3, triple, 0)
            wait_gather(sref, CKB - 3, 1)
            start_scatter(dref, CKB - 3, 1)
            wait_scatter(dref, CKB - 4, 0)
            start_gather(sref, CKB - 1, 0)
            wait_gather(sref, CKB - 2, 2)
            start_scatter(dref, CKB - 2, 2)
            wait_scatter(dref, CKB - 3, 1)
            wait_gather(sref, CKB - 1, 0)
            start_scatter(dref, CKB - 1, 0)
            wait_scatter(dref, CKB - 2, 2)
            wait_scatter(dref, CKB - 1, 0)

        plsc.subcore_barrier()
        pltpu.sync_copy(acc_sh.at[pl.ds(off, RPT)], out_hbm.at[c, s])

    return k(er, h, zrows)


def _norm_col(deg_block):
    return jnp.where(deg_block > 0.0,
                     lax.rsqrt(jnp.maximum(deg_block, 1.0)), 0.0)[:, 0:1]


_RB = 1000


def _tc_layer1(x, W1, degs):
    def body(x_ref, w_ref, ds_ref, o_ref):
        ns = _norm_col(ds_ref[...])
        h = jnp.dot(x_ref[...], w_ref[...], preferred_element_type=jnp.float32)
        o_ref[...] = h * ns

    return pl.pallas_call(
        body,
        grid=(N // _RB,),
        in_specs=[
            pl.BlockSpec((_RB, D), lambda i: (i, 0)),
            pl.BlockSpec((D, D), lambda i: (0, 0)),
            pl.BlockSpec((_RB, D), lambda i: (i, 0)),
        ],
        out_specs=pl.BlockSpec((_RB, D), lambda i: (i, 0)),
        out_shape=jax.ShapeDtypeStruct((N, D), jnp.float32),
    )(x, W1, degs)


def _tc_layer2(a0, a1, degs, degd, b1r, W2):
    def body(a0_ref, a1_ref, ds_ref, dd_ref, b_ref, w_ref, o_ref):
        nd = _norm_col(dd_ref[...])
        ns = _norm_col(ds_ref[...])
        y = jnp.maximum((a0_ref[...] + a1_ref[...]) * nd + b_ref[...], 0.0)
        o_ref[...] = jnp.dot(
            y, w_ref[...], preferred_element_type=jnp.float32) * ns

    return pl.pallas_call(
        body,
        grid=(N // _RB,),
        in_specs=[
            pl.BlockSpec((_RB, D), lambda i: (i, 0)),
            pl.BlockSpec((_RB, D), lambda i: (i, 0)),
            pl.BlockSpec((_RB, D), lambda i: (i, 0)),
            pl.BlockSpec((_RB, D), lambda i: (i, 0)),
            pl.BlockSpec((1, D), lambda i: (0, 0)),
            pl.BlockSpec((D, D), lambda i: (0, 0)),
        ],
        out_specs=pl.BlockSpec((_RB, D), lambda i: (i, 0)),
        out_shape=jax.ShapeDtypeStruct((N, D), jnp.float32),
    )(a0, a1, degs, degd, b1r, W2)


def _tc_final(a0, a1, degd, b2r):
    def body(a0_ref, a1_ref, dd_ref, b_ref, o_ref):
        nd = _norm_col(dd_ref[...])
        o_ref[...] = (a0_ref[...] + a1_ref[...]) * nd + b_ref[...]

    return pl.pallas_call(
        body,
        grid=(N // _RB,),
        in_specs=[
            pl.BlockSpec((_RB, D), lambda i: (i, 0)),
            pl.BlockSpec((_RB, D), lambda i: (i, 0)),
            pl.BlockSpec((_RB, D), lambda i: (i, 0)),
            pl.BlockSpec((1, D), lambda i: (0, 0)),
        ],
        out_specs=pl.BlockSpec((_RB, D), lambda i: (i, 0)),
        out_shape=jax.ShapeDtypeStruct((N, D), jnp.float32),
    )(a0, a1, degd, b2r)


def kernel(x, edge_index, W1, b1, W2, b2):
    er = edge_index.reshape(2, NW, NCK, CKB, EB)
    zrows = jnp.zeros((RPT, D), jnp.float32)
    deg = _sc_degrees(er, zrows).reshape(NC, NPAD, D)[:, :N]
    degs, degd = deg[0], deg[1]
    hs1 = _tc_layer1(x, W1, degs)
    ap1 = _sc_edge_agg(er, hs1, zrows).reshape(NC, NPAD, D)[:, :N]
    hs2 = _tc_layer2(ap1[0], ap1[1], degs, degd, b1.reshape(1, D), W2)
    ap2 = _sc_edge_agg(er, hs2, zrows).reshape(NC, NPAD, D)[:, :N]
    return _tc_final(ap2[0], ap2[1], degd, b2.reshape(1, D))


# speedup vs baseline: 15.5663x; 1.0198x over previous
import functools

import jax
import jax.numpy as jnp
from jax import lax
from jax.experimental import pallas as pl
from jax.experimental.pallas import tpu as pltpu
from jax.experimental.pallas import tpu_sc as plsc

N = 10000
E = 320000
D = 128
NC = 2
NS = 16
NW = NC * NS
LANES = 16
EB = 80
TPR = E // EB // NW
NCK = 5
CKB = TPR // NCK
NPAD = 10240
RPT = NPAD // NS

_MESH = dict(core_axis_name="c", subcore_axis_name="s", num_cores=NC,
             num_subcores=NS)


def _sc_degrees(er, zrows):
    WIN = 8

    @functools.partial(
        pl.kernel,
        out_type=jax.ShapeDtypeStruct((NC, NS, RPT, D), jnp.float32),
        mesh=plsc.VectorSubcoreMesh(**_MESH),
        scratch_types=[
            pltpu.VMEM((CKB, EB), jnp.int32),
            pltpu.VMEM((CKB, EB), jnp.int32),
            pltpu.VMEM((EB, D), jnp.float32),
            pltpu.SemaphoreType.DMA,
            pltpu.SemaphoreType.DMA,
            pltpu.VMEM_SHARED((NPAD, D), jnp.float32),
        ],
    )
    def k(er_hbm, z_hbm, deg_out, idx0, idx1, ones_v, asem, isem, acc_sh):
        c = lax.axis_index("c")
        s = lax.axis_index("s")
        idxb = (idx0, idx1)

        def fill_ones(i, carry):
            for t16 in range(D // LANES):
                ones_v[i, pl.ds(t16 * LANES, LANES)] = jnp.ones(
                    (LANES,), jnp.float32)
            return carry

        lax.fori_loop(0, EB, fill_ones, 0)
        off = pl.multiple_of(s * RPT, RPT)
        pltpu.sync_copy(z_hbm, acc_sh.at[pl.ds(off, RPT)])
        plsc.subcore_barrier()

        def start_add(iref, j):
            pltpu.async_copy(ones_v, acc_sh.at[iref.at[j]], asem, add=True)

        def wait_add(iref):
            pltpu.make_async_copy(ones_v, acc_sh.at[iref.at[0]], asem).wait()

        for t in range(2 * NCK):
            p, q = t // NCK, t % NCK
            tb = t % 2
            iref = idxb[tb]
            if t == 0:
                pltpu.sync_copy(er_hbm.at[c, 2 * s + p, q], iref)
            else:
                pltpu.make_async_copy(er_hbm.at[c, 2 * s + p, q], iref,
                                      isem).wait()
            if t + 1 < 2 * NCK:
                p1, q1 = (t + 1) // NCK, (t + 1) % NCK
                pltpu.async_copy(er_hbm.at[c, 2 * s + p1, q1],
                                 idxb[(t + 1) % 2], isem)
            for j in range(WIN):
                start_add(iref, j)

            def steady(j, carry):
                wait_add(iref)
                start_add(iref, j)
                return carry

            lax.fori_loop(WIN, CKB, steady, 0)
            for _ in range(WIN):
                wait_add(iref)
        plsc.subcore_barrier()
        pltpu.sync_copy(acc_sh.at[pl.ds(off, RPT)], deg_out.at[c, s])

    return k(er, zrows)


def _sc_edge_agg(er, h, zrows):
    @functools.partial(
        pl.kernel,
        out_type=jax.ShapeDtypeStruct((NC, NS, RPT, D), jnp.float32),
        mesh=plsc.VectorSubcoreMesh(**_MESH),
        scratch_types=[
            pltpu.VMEM((CKB, EB), jnp.int32),
            pltpu.VMEM((CKB, EB), jnp.int32),
            pltpu.VMEM((CKB, EB), jnp.int32),
            pltpu.VMEM((CKB, EB), jnp.int32),
            pltpu.VMEM((EB, D), jnp.float32),
            pltpu.VMEM((EB, D), jnp.float32),
            pltpu.VMEM((EB, D), jnp.float32),
            pltpu.SemaphoreType.DMA,
            pltpu.SemaphoreType.DMA,
            pltpu.SemaphoreType.DMA,
            pltpu.SemaphoreType.DMA,
            pltpu.SemaphoreType.DMA,
            pltpu.SemaphoreType.DMA,
            pltpu.SemaphoreType.DMA,
            pltpu.VMEM_SHARED((NPAD, D), jnp.float32),
        ],
    )
    def k(er_hbm, h_hbm, z_hbm, out_hbm, src0, src1, dst0, dst1,
          rows0, rows1, rows2, gs0, gs1, gs2, ss0, ss1, ss2, isem, acc_sh):
        c = lax.axis_index("c")
        s = lax.axis_index("s")
        w = c * NS + s
        srcb, dstb = (src0, src1), (dst0, dst1)
        rows = (rows0, rows1, rows2)
        gsems, ssems = (gs0, gs1, gs2), (ss0, ss1, ss2)

        off = pl.multiple_of(s * RPT, RPT)
        pltpu.sync_copy(z_hbm, acc_sh.at[pl.ds(off, RPT)])
        plsc.subcore_barrier()

        def start_gather(sref, j, b):
            pltpu.async_copy(h_hbm.at[sref.at[j]], rows[b], gsems[b])

        def wait_gather(sref, j, b):
            pltpu.make_async_copy(h_hbm.at[sref.at[j]], rows[b],
                                  gsems[b]).wait()

        def start_scatter(dref, j, b):
            pltpu.async_copy(rows[b], acc_sh.at[dref.at[j]], ssems[b],
                             add=True)

        def wait_scatter(dref, j, b):
            pltpu.make_async_copy(rows[b], acc_sh.at[dref.at[j]],
                                  ssems[b]).wait()

        for q in range(NCK):
            qb = q % 2
            sref, dref = srcb[qb], dstb[qb]
            if q == 0:
                pltpu.sync_copy(er_hbm.at[0, w, 0], sref)
                pltpu.sync_copy(er_hbm.at[1, w, 0], dref)
            else:
                pltpu.make_async_copy(er_hbm.at[0, w, q], sref, isem).wait()
                pltpu.make_async_copy(er_hbm.at[1, w, q], dref, isem).wait()
            if q + 1 < NCK:
                nb = (q + 1) % 2
                pltpu.async_copy(er_hbm.at[0, w, q + 1], srcb[nb], isem)
                pltpu.async_copy(er_hbm.at[1, w, q + 1], dstb[nb], isem)

            start_gather(sref, 0, 0)
            start_gather(sref, 1, 1)
            wait_gather(sref, 0, 0)
            start_scatter(dref, 0, 0)
            start_gather(sref, 2, 2)

            def triple(jj, carry):
                for b in (1, 2, 0):
                    j = 3 * jj + (1 if b == 1 else (2 if b == 2 else 3))
                    wait_gather(sref, j, b)
                    start_scatter(dref, j, b)
                    wait_scatter(dref, j - 1, (b + 2) % 3)
                    start_gather(sref, j + 2, (b + 2) % 3)
                return carry

            lax.fori_loop(0, (CKB - 4) // 3, triple, 0)
            wait_gather(sref, CKB - 3, 1)
            start_scatter(dref, CKB - 3, 1)
            wait_scatter(dref, CKB - 4, 0)
            start_gather(sref, CKB - 1, 0)
            wait_gather(sref, CKB - 2, 2)
            start_scatter(dref, CKB - 2, 2)
            wait_scatter(dref, CKB - 3, 1)
            wait_gather(sref, CKB - 1, 0)
            start_scatter(dref, CKB - 1, 0)
            wait_scatter(dref, CKB - 2, 2)
            wait_scatter(dref, CKB - 1, 0)

        plsc.subcore_barrier()
        pltpu.sync_copy(acc_sh.at[pl.ds(off, RPT)], out_hbm.at[c, s])

    return k(er, h, zrows)


def _norm_col(deg_block):
    return jnp.where(deg_block > 0.0,
                     lax.rsqrt(jnp.maximum(deg_block, 1.0)), 0.0)[:, 0:1]


_RB = 1000


def _tc_layer1(x, W1, degs):
    def body(x_ref, w_ref, ds_ref, o_ref):
        ns = _norm_col(ds_ref[...])
        h = jnp.dot(x_ref[...], w_ref[...], preferred_element_type=jnp.float32)
        o_ref[...] = h * ns

    return pl.pallas_call(
        body,
        grid=(N // _RB,),
        in_specs=[
            pl.BlockSpec((_RB, D), lambda i: (i, 0)),
            pl.BlockSpec((D, D), lambda i: (0, 0)),
            pl.BlockSpec((_RB, D), lambda i: (i, 0)),
        ],
        out_specs=pl.BlockSpec((_RB, D), lambda i: (i, 0)),
        out_shape=jax.ShapeDtypeStruct((N, D), jnp.float32),
    )(x, W1, degs)


def _tc_layer2(a0, a1, degs, degd, b1r, W2):
    def body(a0_ref, a1_ref, ds_ref, dd_ref, b_ref, w_ref, o_ref):
        nd = _norm_col(dd_ref[...])
        ns = _norm_col(ds_ref[...])
        y = jnp.maximum((a0_ref[...] + a1_ref[...]) * nd + b_ref[...], 0.0)
        o_ref[...] = jnp.dot(
            y, w_ref[...], preferred_element_type=jnp.float32) * ns

    return pl.pallas_call(
        body,
        grid=(N // _RB,),
        in_specs=[
            pl.BlockSpec((_RB, D), lambda i: (i, 0)),
            pl.BlockSpec((_RB, D), lambda i: (i, 0)),
            pl.BlockSpec((_RB, D), lambda i: (i, 0)),
            pl.BlockSpec((_RB, D), lambda i: (i, 0)),
            pl.BlockSpec((1, D), lambda i: (0, 0)),
            pl.BlockSpec((D, D), lambda i: (0, 0)),
        ],
        out_specs=pl.BlockSpec((_RB, D), lambda i: (i, 0)),
        out_shape=jax.ShapeDtypeStruct((N, D), jnp.float32),
    )(a0, a1, degs, degd, b1r, W2)


def _tc_final(a0, a1, degd, b2r):
    def body(a0_ref, a1_ref, dd_ref, b_ref, o_ref):
        nd = _norm_col(dd_ref[...])
        o_ref[...] = (a0_ref[...] + a1_ref[...]) * nd + b_ref[...]

    return pl.pallas_call(
        body,
        grid=(N // _RB,),
        in_specs=[
            pl.BlockSpec((_RB, D), lambda i: (i, 0)),
            pl.BlockSpec((_RB, D), lambda i: (i, 0)),
            pl.BlockSpec((_RB, D), lambda i: (i, 0)),
            pl.BlockSpec((1, D), lambda i: (0, 0)),
        ],
        out_specs=pl.BlockSpec((_RB, D), lambda i: (i, 0)),
        out_shape=jax.ShapeDtypeStruct((N, D), jnp.float32),
    )(a0, a1, degd, b2r)


def kernel(x, edge_index, W1, b1, W2, b2):
    er = edge_index.reshape(2, NW, NCK, CKB, EB)
    zrows = jnp.zeros((RPT, D), jnp.float32)
    deg = _sc_degrees(er, zrows).reshape(NC, NPAD, D)[:, :N]
    degs, degd = deg[0], deg[1]
    hs1 = _tc_layer1(x, W1, degs)
    ap1 = _sc_edge_agg(er, hs1, zrows).reshape(NC, NPAD, D)[:, :N]
    hs2 = _tc_layer2(ap1[0], ap1[1], degs, degd, b1.reshape(1, D), W2)
    ap2 = _sc_edge_agg(er, hs2, zrows).reshape(NC, NPAD, D)[:, :N]
    return _tc_final(ap2[0], ap2[1], degd, b2.reshape(1, D))
